# baseline (speedup 1.0000x reference)
import functools

import jax
import jax.numpy as jnp
from jax import lax
from jax.experimental import pallas as pl
from jax.experimental.pallas import tpu as pltpu
from jax.experimental.pallas import tpu_sc as plsc

_NC = 2
_NS = 16
_NW = _NC * _NS

_IB = 128
_G = 4


@functools.lru_cache(maxsize=None)
def _build(n_blocks, vocab, d):
  blks_per_w = n_blocks // _NW
  n_chunks = blks_per_w // _G
  mesh = plsc.VectorSubcoreMesh(
      core_axis_name="c", subcore_axis_name="s",
      num_cores=_NC, num_subcores=_NS)

  @functools.partial(
      pl.kernel,
      out_type=jax.ShapeDtypeStruct((n_blocks, _IB, d), jnp.float32),
      mesh=mesh,
      scratch_types=[
          pltpu.VMEM((_G, _IB), jnp.int32),
          pltpu.VMEM((_G, _IB, d), jnp.float32),
          pltpu.SemaphoreType.DMA,
      ],
      compiler_params=pltpu.CompilerParams(use_tc_tiling_on_sc=False),
  )
  def lookup(idx_hbm, table_hbm, out_hbm, idx_v, rows_v, gsem):
    wid = lax.axis_index("s") * _NC + lax.axis_index("c")
    blk0 = wid * blks_per_w

    def chunk(ci, carry):
      rb = blk0 + ci * _G
      pltpu.sync_copy(idx_hbm.at[pl.ds(rb, _G)], idx_v)
      descs = [
          pltpu.make_async_copy(
              table_hbm.at[idx_v.at[j]], rows_v.at[j], gsem)
          for j in range(_G)
      ]
      for dsc in descs:
        dsc.start()
      for dsc in descs:
        dsc.wait()
      pltpu.sync_copy(rows_v, out_hbm.at[pl.ds(rb, _G)])
      return carry

    lax.fori_loop(0, n_chunks, chunk, 0)

  return lookup


def kernel(inputs, embeddings):
  b, h = inputs.shape
  vocab, d = embeddings.shape
  n = b * h
  idx2d = jnp.reshape(inputs.astype(jnp.int32), (n // _IB, _IB))
  out = _build(n // _IB, vocab, d)(idx2d, embeddings)
  return jnp.reshape(out, (b, h, d))


# speedup vs baseline: 1.0347x; 1.0347x over previous
import functools

import jax
import jax.numpy as jnp
from jax import lax
from jax.experimental import pallas as pl
from jax.experimental.pallas import tpu as pltpu
from jax.experimental.pallas import tpu_sc as plsc

_NC = 2
_NS = 16
_NW = _NC * _NS

_IB = 128
_G = 4


@functools.lru_cache(maxsize=None)
def _build(n_blocks, vocab, d):
  blks_per_w = n_blocks // _NW
  n_chunks = blks_per_w // _G
  mesh = plsc.VectorSubcoreMesh(
      core_axis_name="c", subcore_axis_name="s",
      num_cores=_NC, num_subcores=_NS)

  @functools.partial(
      pl.kernel,
      out_type=jax.ShapeDtypeStruct((n_blocks, _IB, d), jnp.float32),
      mesh=mesh,
      scratch_types=[
          pltpu.VMEM((2, _G, _IB), jnp.int32),
          pltpu.VMEM((2, _G, _IB, d), jnp.float32),
          pltpu.SemaphoreType.DMA,
      ],
      compiler_params=pltpu.CompilerParams(use_tc_tiling_on_sc=False),
  )
  def lookup(idx_hbm, table_hbm, out_hbm, idx_v, rows_v, gsem):
    wid = lax.axis_index("s") * _NC + lax.axis_index("c")
    blk0 = wid * blks_per_w

    def fire(k, s):
      rb = blk0 + k * _G
      pltpu.sync_copy(idx_hbm.at[pl.ds(rb, _G)], idx_v.at[s])
      for j in range(_G):
        pltpu.make_async_copy(
            table_hbm.at[idx_v.at[s, j]], rows_v.at[s, j], gsem).start()

    def drain_store(k, s):
      for j in range(_G):
        pltpu.make_async_copy(
            table_hbm.at[idx_v.at[s, j]], rows_v.at[s, j], gsem).wait()
      pltpu.sync_copy(rows_v.at[s], out_hbm.at[pl.ds(blk0 + k * _G, _G)])

    fire(0, 0)
    fire(1, 1)

    def body(i, carry):
      for b in range(2):
        k = i * 2 + b
        drain_store(k, b)

        @pl.when(k + 2 < n_chunks)
        def _():
          fire(k + 2, b)
      return carry

    lax.fori_loop(0, n_chunks // 2, body, 0)

  return lookup


def kernel(inputs, embeddings):
  b, h = inputs.shape
  vocab, d = embeddings.shape
  n = b * h
  idx2d = jnp.reshape(inputs.astype(jnp.int32), (n // _IB, _IB))
  out = _build(n // _IB, vocab, d)(idx2d, embeddings)
  return jnp.reshape(out, (b, h, d))
